# baseline (speedup 1.0000x reference)
import functools

import jax
import jax.numpy as jnp
from jax import lax
from jax.experimental import pallas as pl
from jax.experimental.pallas import tpu as pltpu
from jax.experimental.pallas import tpu_sc as plsc

N_NODES = 10000
D = 128
E = 320000

NC = 2
NS = 16
NW = NC * NS

B = 128
NPAD = 10240
PAD_IDX = N_NODES
NB0 = 122
NB1 = 36
NBT = NB0 + NB1
EPAD = NS * NBT * B
NPT = NPAD // NS

RB = 1024
GRID = NPAD // RB

_mesh = plsc.VectorSubcoreMesh(core_axis_name="c", subcore_axis_name="s")


@functools.partial(
    pl.kernel,
    out_type=(
        jax.ShapeDtypeStruct((NPAD,), jnp.float32),
        jax.ShapeDtypeStruct((NPAD,), jnp.float32),
    ),
    mesh=_mesh,
    scratch_types=(
        pltpu.VMEM_SHARED((NPAD,), jnp.float32),
        pltpu.VMEM((2, B), jnp.int32),
        pltpu.VMEM((B,), jnp.float32),
    ),
)
def _deg_kernel(edges_hbm, ones_hbm, zeros_hbm, out0, out1, acc_sh, cidx,
                ones_v):
    c = lax.axis_index("c")
    s = lax.axis_index("s")
    w = s * NC + c
    lo = s * NPT

    @pl.when(c == 0)
    def _():
        pltpu.sync_copy(ones_hbm.at[pl.ds(lo, NPT)], acc_sh.at[pl.ds(lo, NPT)])

    @pl.when(c != 0)
    def _():
        pltpu.sync_copy(zeros_hbm.at[pl.ds(lo, NPT)], acc_sh.at[pl.ds(lo, NPT)])

    pltpu.sync_copy(ones_hbm.at[pl.ds(0, B)], ones_v)
    plsc.subcore_barrier()

    nb = NS * NBT // NW
    tbase = w * nb

    def body(j, carry):
        pltpu.sync_copy(edges_hbm.at[tbase + j], cidx)
        pltpu.sync_copy(ones_v, acc_sh.at[cidx.at[1]], add=True)
        return carry

    lax.fori_loop(0, nb, body, 0)
    plsc.subcore_barrier()

    @pl.when(c == 0)
    def _():
        pltpu.sync_copy(acc_sh.at[pl.ds(lo, NPT)], out0.at[pl.ds(lo, NPT)])

    @pl.when(c != 0)
    def _():
        pltpu.sync_copy(acc_sh.at[pl.ds(lo, NPT)], out1.at[pl.ds(lo, NPT)])


@functools.partial(
    pl.kernel,
    out_type=(
        jax.ShapeDtypeStruct((NPAD, D), jnp.float32),
        jax.ShapeDtypeStruct((NPAD, D), jnp.float32),
    ),
    mesh=_mesh,
    scratch_types=(
        pltpu.VMEM_SHARED((NPAD, D), jnp.float32),
        tuple(pltpu.VMEM((2, B), jnp.int32) for _ in range(2)),
        tuple(pltpu.VMEM((B, D), jnp.float32) for _ in range(2)),
        tuple(pltpu.SemaphoreType.DMA for _ in range(2)),
    ),
)
def _agg_kernel(h_hbm, edges_hbm, zeros_hbm, out0, out1, acc_sh,
                idx, rows, gsems):
    c = lax.axis_index("c")
    s = lax.axis_index("s")
    w = s * NC + c
    lo = s * NPT

    @pl.when(c == 0)
    def _():
        pltpu.sync_copy(h_hbm.at[pl.ds(lo, NPT)], acc_sh.at[pl.ds(lo, NPT)])

    @pl.when(c != 0)
    def _():
        pltpu.sync_copy(zeros_hbm.at[pl.ds(lo, NPT)], acc_sh.at[pl.ds(lo, NPT)])

    plsc.subcore_barrier()

    def run_edges(tbase, nb):
        for b in range(2):
            pltpu.sync_copy(edges_hbm.at[tbase + b], idx[b])
            pltpu.async_copy(h_hbm.at[idx[b].at[0]], rows[b], gsems[b])

        def body(i, carry):
            j = i * 2
            for b in range(2):
                pltpu.make_async_copy(
                    h_hbm.at[idx[b].at[0]], rows[b], gsems[b]).wait()
                pltpu.sync_copy(rows[b], acc_sh.at[idx[b].at[1]], add=True)

                @pl.when(j + b + 2 < nb)
                def _():
                    pltpu.sync_copy(edges_hbm.at[tbase + j + b + 2], idx[b])
                    pltpu.async_copy(h_hbm.at[idx[b].at[0]], rows[b], gsems[b])
            return carry

        lax.fori_loop(0, nb // 2, body, 0)

    @pl.when(c == 0)
    def _():
        run_edges(s * NB0, NB0)

    @pl.when(c != 0)
    def _():
        run_edges(NS * NB0 + s * NB1, NB1)

    plsc.subcore_barrier()

    @pl.when(c == 0)
    def _():
        pltpu.sync_copy(acc_sh.at[pl.ds(lo, NPT)], out0.at[pl.ds(lo, NPT)])

    @pl.when(c != 0)
    def _():
        pltpu.sync_copy(acc_sh.at[pl.ds(lo, NPT)], out1.at[pl.ds(lo, NPT)])


def _mm_body(x_ref, w_ref, o_ref):
    o_ref[...] = jnp.dot(x_ref[...], w_ref[...],
                         preferred_element_type=jnp.float32)


_mm_call = pl.pallas_call(
    _mm_body,
    grid=(GRID,),
    in_specs=[
        pl.BlockSpec((RB, D), lambda i: (i, 0)),
        pl.BlockSpec((D, D), lambda i: (0, 0)),
    ],
    out_specs=pl.BlockSpec((RB, D), lambda i: (i, 0)),
    out_shape=jax.ShapeDtypeStruct((NPAD, D), jnp.float32),
)


def _scale_body(d0_ref, d1_ref, xw_ref, dinv_ref, h_ref):
    deg = d0_ref[...] + d1_ref[...]
    dinv = lax.rsqrt(deg)
    dinv_ref[...] = dinv
    h_ref[...] = xw_ref[...] * dinv[:, None]


_scale_call = pl.pallas_call(
    _scale_body,
    grid=(GRID,),
    in_specs=[
        pl.BlockSpec((RB,), lambda i: (i,)),
        pl.BlockSpec((RB,), lambda i: (i,)),
        pl.BlockSpec((RB, D), lambda i: (i, 0)),
    ],
    out_specs=[
        pl.BlockSpec((RB,), lambda i: (i,)),
        pl.BlockSpec((RB, D), lambda i: (i, 0)),
    ],
    out_shape=[
        jax.ShapeDtypeStruct((NPAD,), jnp.float32),
        jax.ShapeDtypeStruct((NPAD, D), jnp.float32),
    ],
)


def _fused_body(a0_ref, a1_ref, dinv_ref, b_ref, w_ref, o_ref):
    dinv = dinv_ref[...]
    t = (a0_ref[...] + a1_ref[...]) * dinv[:, None] + b_ref[...]
    t = jnp.maximum(t, 0.0)
    z = jnp.dot(t, w_ref[...], preferred_element_type=jnp.float32)
    o_ref[...] = z * dinv[:, None]


_fused_call = pl.pallas_call(
    _fused_body,
    grid=(GRID,),
    in_specs=[
        pl.BlockSpec((RB, D), lambda i: (i, 0)),
        pl.BlockSpec((RB, D), lambda i: (i, 0)),
        pl.BlockSpec((RB,), lambda i: (i,)),
        pl.BlockSpec((1, D), lambda i: (0, 0)),
        pl.BlockSpec((D, D), lambda i: (0, 0)),
    ],
    out_specs=pl.BlockSpec((RB, D), lambda i: (i, 0)),
    out_shape=jax.ShapeDtypeStruct((NPAD, D), jnp.float32),
)


def _final_body(a0_ref, a1_ref, dinv_ref, b_ref, o_ref):
    o_ref[...] = ((a0_ref[...] + a1_ref[...]) * dinv_ref[...][:, None]
                  + b_ref[...])


_final_call = pl.pallas_call(
    _final_body,
    grid=(GRID,),
    in_specs=[
        pl.BlockSpec((RB, D), lambda i: (i, 0)),
        pl.BlockSpec((RB, D), lambda i: (i, 0)),
        pl.BlockSpec((RB,), lambda i: (i,)),
        pl.BlockSpec((1, D), lambda i: (0, 0)),
    ],
    out_specs=pl.BlockSpec((RB, D), lambda i: (i, 0)),
    out_shape=jax.ShapeDtypeStruct((NPAD, D), jnp.float32),
)


def kernel(x, edge_index, W1, b1, W2, b2):
    row = edge_index[0].astype(jnp.int32)
    col = edge_index[1].astype(jnp.int32)
    pad = EPAD - E
    rowp = jnp.concatenate([row, jnp.full((pad,), PAD_IDX, jnp.int32)])
    colp = jnp.concatenate([col, jnp.full((pad,), PAD_IDX, jnp.int32)])
    edges = jnp.stack([rowp.reshape(-1, B), colp.reshape(-1, B)], axis=1)
    xp = jnp.pad(x, ((0, NPAD - N_NODES), (0, 0)))
    zeros2d = jnp.zeros((NPAD, D), jnp.float32)
    ones1d = jnp.ones((NPAD,), jnp.float32)
    zeros1d = jnp.zeros((NPAD,), jnp.float32)
    b1r = b1.reshape(1, D)
    b2r = b2.reshape(1, D)

    deg0, deg1 = _deg_kernel(edges, ones1d, zeros1d)
    xw1 = _mm_call(xp, W1)
    dinv, h1s = _scale_call(deg0, deg1, xw1)

    a0, a1 = _agg_kernel(h1s, edges, zeros2d)
    h2s = _fused_call(a0, a1, dinv, b1r, W2)
    c0, c1 = _agg_kernel(h2s, edges, zeros2d)
    out = _final_call(c0, c1, dinv, b2r)
    return out[:N_NODES]


# speedup vs baseline: 1.0675x; 1.0675x over previous
import functools

import jax
import jax.numpy as jnp
from jax import lax
from jax.experimental import pallas as pl
from jax.experimental.pallas import tpu as pltpu
from jax.experimental.pallas import tpu_sc as plsc

N_NODES = 10000
D = 128
E = 320000

NC = 2
NS = 16
NW = NC * NS

B = 128
NPAD = 10240
PAD_IDX = N_NODES
NB0 = 122
NB1 = 36
NBT = NB0 + NB1
EPAD = NS * NBT * B
NPT = NPAD // NS

RB = 1024
GRID = NPAD // RB

_mesh = plsc.VectorSubcoreMesh(core_axis_name="c", subcore_axis_name="s")


@functools.partial(
    pl.kernel,
    out_type=(
        jax.ShapeDtypeStruct((NPAD,), jnp.float32),
        jax.ShapeDtypeStruct((NPAD,), jnp.float32),
    ),
    mesh=_mesh,
    scratch_types=(
        pltpu.VMEM_SHARED((NPAD,), jnp.float32),
        pltpu.VMEM((B,), jnp.int32),
        pltpu.VMEM((B,), jnp.float32),
    ),
)
def _deg_kernel(col_hbm, ones_hbm, zeros_hbm, out0, out1, acc_sh, cidx,
                ones_v):
    c = lax.axis_index("c")
    s = lax.axis_index("s")
    w = s * NC + c
    lo = s * NPT

    @pl.when(c == 0)
    def _():
        pltpu.sync_copy(ones_hbm.at[pl.ds(lo, NPT)], acc_sh.at[pl.ds(lo, NPT)])

    @pl.when(c != 0)
    def _():
        pltpu.sync_copy(zeros_hbm.at[pl.ds(lo, NPT)], acc_sh.at[pl.ds(lo, NPT)])

    pltpu.sync_copy(ones_hbm.at[pl.ds(0, B)], ones_v)
    plsc.subcore_barrier()

    nb = NS * NBT // NW
    base = w * nb * B

    def body(j, carry):
        pltpu.sync_copy(col_hbm.at[pl.ds(base + j * B, B)], cidx)
        pltpu.sync_copy(ones_v, acc_sh.at[cidx], add=True)
        return carry

    lax.fori_loop(0, nb, body, 0)
    plsc.subcore_barrier()

    @pl.when(c == 0)
    def _():
        pltpu.sync_copy(acc_sh.at[pl.ds(lo, NPT)], out0.at[pl.ds(lo, NPT)])

    @pl.when(c != 0)
    def _():
        pltpu.sync_copy(acc_sh.at[pl.ds(lo, NPT)], out1.at[pl.ds(lo, NPT)])


@functools.partial(
    pl.kernel,
    out_type=(
        jax.ShapeDtypeStruct((NPAD, D), jnp.float32),
        jax.ShapeDtypeStruct((NPAD, D), jnp.float32),
    ),
    mesh=_mesh,
    scratch_types=(
        pltpu.VMEM_SHARED((NPAD, D), jnp.float32),
        tuple(pltpu.VMEM((B,), jnp.int32) for _ in range(2)),
        tuple(pltpu.VMEM((B,), jnp.int32) for _ in range(2)),
        tuple(pltpu.VMEM((B, D), jnp.float32) for _ in range(2)),
        tuple(pltpu.SemaphoreType.DMA for _ in range(2)),
    ),
)
def _agg_kernel(h_hbm, row_hbm, col_hbm, zeros_hbm, out0, out1, acc_sh,
                ridx, cidx, rows, gsems):
    c = lax.axis_index("c")
    s = lax.axis_index("s")
    w = s * NC + c
    lo = s * NPT

    @pl.when(c == 0)
    def _():
        pltpu.sync_copy(h_hbm.at[pl.ds(lo, NPT)], acc_sh.at[pl.ds(lo, NPT)])

    @pl.when(c != 0)
    def _():
        pltpu.sync_copy(zeros_hbm.at[pl.ds(lo, NPT)], acc_sh.at[pl.ds(lo, NPT)])

    plsc.subcore_barrier()

    def run_edges(base, nb):
        for b in range(2):
            pltpu.sync_copy(row_hbm.at[pl.ds(base + b * B, B)], ridx[b])
            pltpu.async_copy(h_hbm.at[ridx[b]], rows[b], gsems[b])

        def body(i, carry):
            j = i * 2
            for b in range(2):
                pltpu.sync_copy(
                    col_hbm.at[pl.ds(base + (j + b) * B, B)], cidx[b])
                pltpu.make_async_copy(
                    h_hbm.at[ridx[b]], rows[b], gsems[b]).wait()
                pltpu.sync_copy(rows[b], acc_sh.at[cidx[b]], add=True)

                @pl.when(j + b + 2 < nb)
                def _():
                    pltpu.sync_copy(
                        row_hbm.at[pl.ds(base + (j + b + 2) * B, B)], ridx[b])
                    pltpu.async_copy(h_hbm.at[ridx[b]], rows[b], gsems[b])
            return carry

        lax.fori_loop(0, nb // 2, body, 0)

    @pl.when(c == 0)
    def _():
        run_edges(s * (NB0 * B), NB0)

    @pl.when(c != 0)
    def _():
        run_edges(NS * (NB0 * B) + s * (NB1 * B), NB1)

    plsc.subcore_barrier()

    @pl.when(c == 0)
    def _():
        pltpu.sync_copy(acc_sh.at[pl.ds(lo, NPT)], out0.at[pl.ds(lo, NPT)])

    @pl.when(c != 0)
    def _():
        pltpu.sync_copy(acc_sh.at[pl.ds(lo, NPT)], out1.at[pl.ds(lo, NPT)])


def _mm_body(x_ref, w_ref, o_ref):
    o_ref[...] = jnp.dot(x_ref[...], w_ref[...],
                         preferred_element_type=jnp.float32)


_mm_call = pl.pallas_call(
    _mm_body,
    grid=(GRID,),
    in_specs=[
        pl.BlockSpec((RB, D), lambda i: (i, 0)),
        pl.BlockSpec((D, D), lambda i: (0, 0)),
    ],
    out_specs=pl.BlockSpec((RB, D), lambda i: (i, 0)),
    out_shape=jax.ShapeDtypeStruct((NPAD, D), jnp.float32),
)


def _scale_body(d0_ref, d1_ref, xw_ref, dinv_ref, h_ref):
    deg = d0_ref[...] + d1_ref[...]
    dinv = lax.rsqrt(deg)
    dinv_ref[...] = dinv
    h_ref[...] = xw_ref[...] * dinv[:, None]


_scale_call = pl.pallas_call(
    _scale_body,
    grid=(GRID,),
    in_specs=[
        pl.BlockSpec((RB,), lambda i: (i,)),
        pl.BlockSpec((RB,), lambda i: (i,)),
        pl.BlockSpec((RB, D), lambda i: (i, 0)),
    ],
    out_specs=[
        pl.BlockSpec((RB,), lambda i: (i,)),
        pl.BlockSpec((RB, D), lambda i: (i, 0)),
    ],
    out_shape=[
        jax.ShapeDtypeStruct((NPAD,), jnp.float32),
        jax.ShapeDtypeStruct((NPAD, D), jnp.float32),
    ],
)


def _fused_body(a0_ref, a1_ref, dinv_ref, b_ref, w_ref, o_ref):
    dinv = dinv_ref[...]
    t = (a0_ref[...] + a1_ref[...]) * dinv[:, None] + b_ref[...]
    t = jnp.maximum(t, 0.0)
    z = jnp.dot(t, w_ref[...], preferred_element_type=jnp.float32)
    o_ref[...] = z * dinv[:, None]


_fused_call = pl.pallas_call(
    _fused_body,
    grid=(GRID,),
    in_specs=[
        pl.BlockSpec((RB, D), lambda i: (i, 0)),
        pl.BlockSpec((RB, D), lambda i: (i, 0)),
        pl.BlockSpec((RB,), lambda i: (i,)),
        pl.BlockSpec((1, D), lambda i: (0, 0)),
        pl.BlockSpec((D, D), lambda i: (0, 0)),
    ],
    out_specs=pl.BlockSpec((RB, D), lambda i: (i, 0)),
    out_shape=jax.ShapeDtypeStruct((NPAD, D), jnp.float32),
)


def _final_body(a0_ref, a1_ref, dinv_ref, b_ref, o_ref):
    o_ref[...] = ((a0_ref[...] + a1_ref[...]) * dinv_ref[...][:, None]
                  + b_ref[...])


_final_call = pl.pallas_call(
    _final_body,
    grid=(GRID,),
    in_specs=[
        pl.BlockSpec((RB, D), lambda i: (i, 0)),
        pl.BlockSpec((RB, D), lambda i: (i, 0)),
        pl.BlockSpec((RB,), lambda i: (i,)),
        pl.BlockSpec((1, D), lambda i: (0, 0)),
    ],
    out_specs=pl.BlockSpec((RB, D), lambda i: (i, 0)),
    out_shape=jax.ShapeDtypeStruct((NPAD, D), jnp.float32),
)


def kernel(x, edge_index, W1, b1, W2, b2):
    row = edge_index[0].astype(jnp.int32)
    col = edge_index[1].astype(jnp.int32)
    pad = EPAD - E
    rowp = jnp.concatenate([row, jnp.full((pad,), PAD_IDX, jnp.int32)])
    colp = jnp.concatenate([col, jnp.full((pad,), PAD_IDX, jnp.int32)])
    xp = jnp.pad(x, ((0, NPAD - N_NODES), (0, 0)))
    zeros2d = jnp.zeros((NPAD, D), jnp.float32)
    ones1d = jnp.ones((NPAD,), jnp.float32)
    zeros1d = jnp.zeros((NPAD,), jnp.float32)
    b1r = b1.reshape(1, D)
    b2r = b2.reshape(1, D)

    deg0, deg1 = _deg_kernel(colp, ones1d, zeros1d)
    xw1 = _mm_call(xp, W1)
    dinv, h1s = _scale_call(deg0, deg1, xw1)

    a0, a1 = _agg_kernel(h1s, rowp, colp, zeros2d)
    h2s = _fused_call(a0, a1, dinv, b1r, W2)
    c0, c1 = _agg_kernel(h2s, rowp, colp, zeros2d)
    out = _final_call(c0, c1, dinv, b2r)
    return out[:N_NODES]
